# baseline (speedup 1.0000x reference)
import jax
import jax.numpy as jnp
from jax.experimental import pallas as pl
from jax.experimental.pallas import tpu as pltpu

_BM = 200


def _xw_kernel(x_ref, w_ref, o_ref):
    o_ref[...] = jnp.dot(x_ref[...], w_ref[...],
                         preferred_element_type=jnp.float32)


def _pass1_kernel(l_ref, s1_ref, b1_ref, w2_ref, o_ref):
    h = jnp.dot(l_ref[...], s1_ref[...], preferred_element_type=jnp.float32)
    h = jnp.maximum(h + b1_ref[...], 0.0)
    o_ref[...] = jnp.dot(h, w2_ref[...], preferred_element_type=jnp.float32)


def _pass2_kernel(l_ref, s2_ref, b2_ref, o_ref):
    o_ref[...] = jnp.dot(l_ref[...], s2_ref[...],
                         preferred_element_type=jnp.float32) + b2_ref[...]


def kernel(Laplacian, feature, W1, b1, W2, b2):
    n, in_dim = feature.shape
    n_hid = W1.shape[1]
    out_dim = W2.shape[1]
    b1r = b1.reshape(1, n_hid)
    b2r = b2.reshape(1, out_dim)

    bx = 2000
    s1 = pl.pallas_call(
        _xw_kernel,
        grid=(n // bx,),
        in_specs=[
            pl.BlockSpec((bx, in_dim), lambda i: (i, 0)),
            pl.BlockSpec((in_dim, n_hid), lambda i: (0, 0)),
        ],
        out_specs=pl.BlockSpec((bx, n_hid), lambda i: (i, 0)),
        out_shape=jax.ShapeDtypeStruct((n, n_hid), jnp.float32),
        compiler_params=pltpu.CompilerParams(
            dimension_semantics=("arbitrary",)),
    )(feature, W1)

    s2 = pl.pallas_call(
        _pass1_kernel,
        grid=(n // _BM,),
        in_specs=[
            pl.BlockSpec((_BM, n), lambda i: (i, 0)),
            pl.BlockSpec((n, n_hid), lambda i: (0, 0)),
            pl.BlockSpec((1, n_hid), lambda i: (0, 0)),
            pl.BlockSpec((n_hid, out_dim), lambda i: (0, 0)),
        ],
        out_specs=pl.BlockSpec((_BM, out_dim), lambda i: (i, 0)),
        out_shape=jax.ShapeDtypeStruct((n, out_dim), jnp.float32),
        compiler_params=pltpu.CompilerParams(
            dimension_semantics=("arbitrary",)),
    )(Laplacian, s1, b1r, W2)

    logits = pl.pallas_call(
        _pass2_kernel,
        grid=(n // _BM,),
        in_specs=[
            pl.BlockSpec((_BM, n), lambda i: (i, 0)),
            pl.BlockSpec((n, out_dim), lambda i: (0, 0)),
            pl.BlockSpec((1, out_dim), lambda i: (0, 0)),
        ],
        out_specs=pl.BlockSpec((_BM, out_dim), lambda i: (i, 0)),
        out_shape=jax.ShapeDtypeStruct((n, out_dim), jnp.float32),
        compiler_params=pltpu.CompilerParams(
            dimension_semantics=("arbitrary",)),
    )(Laplacian, s2, b2r)

    return logits


# speedup vs baseline: 1.0231x; 1.0231x over previous
import jax
import jax.numpy as jnp
from jax.experimental import pallas as pl
from jax.experimental.pallas import tpu as pltpu

_BM = 400


def _xw_kernel(x_ref, w_ref, o_ref):
    o_ref[...] = jnp.dot(x_ref[...], w_ref[...],
                         preferred_element_type=jnp.float32)


def _pass1_kernel(l_ref, s1_ref, b1_ref, w2_ref, o_ref):
    h = jnp.dot(l_ref[...], s1_ref[...], preferred_element_type=jnp.float32)
    h = jnp.maximum(h + b1_ref[...], 0.0)
    o_ref[...] = jnp.dot(h, w2_ref[...], preferred_element_type=jnp.float32)


def _pass2_kernel(l_ref, s2_ref, b2_ref, o_ref):
    o_ref[...] = jnp.dot(l_ref[...], s2_ref[...],
                         preferred_element_type=jnp.float32) + b2_ref[...]


def kernel(Laplacian, feature, W1, b1, W2, b2):
    n, in_dim = feature.shape
    n_hid = W1.shape[1]
    out_dim = W2.shape[1]
    b1r = b1.reshape(1, n_hid)
    b2r = b2.reshape(1, out_dim)

    bx = 2000
    s1 = pl.pallas_call(
        _xw_kernel,
        grid=(n // bx,),
        in_specs=[
            pl.BlockSpec((bx, in_dim), lambda i: (i, 0)),
            pl.BlockSpec((in_dim, n_hid), lambda i: (0, 0)),
        ],
        out_specs=pl.BlockSpec((bx, n_hid), lambda i: (i, 0)),
        out_shape=jax.ShapeDtypeStruct((n, n_hid), jnp.float32),
        compiler_params=pltpu.CompilerParams(
            dimension_semantics=("arbitrary",)),
    )(feature, W1)

    s2 = pl.pallas_call(
        _pass1_kernel,
        grid=(n // _BM,),
        in_specs=[
            pl.BlockSpec((_BM, n), lambda i: (i, 0)),
            pl.BlockSpec((n, n_hid), lambda i: (0, 0)),
            pl.BlockSpec((1, n_hid), lambda i: (0, 0)),
            pl.BlockSpec((n_hid, out_dim), lambda i: (0, 0)),
        ],
        out_specs=pl.BlockSpec((_BM, out_dim), lambda i: (i, 0)),
        out_shape=jax.ShapeDtypeStruct((n, out_dim), jnp.float32),
        compiler_params=pltpu.CompilerParams(
            dimension_semantics=("arbitrary",)),
    )(Laplacian, s1, b1r, W2)

    logits = pl.pallas_call(
        _pass2_kernel,
        grid=(n // _BM,),
        in_specs=[
            pl.BlockSpec((_BM, n), lambda i: (i, 0)),
            pl.BlockSpec((n, out_dim), lambda i: (0, 0)),
            pl.BlockSpec((1, out_dim), lambda i: (0, 0)),
        ],
        out_specs=pl.BlockSpec((_BM, out_dim), lambda i: (i, 0)),
        out_shape=jax.ShapeDtypeStruct((n, out_dim), jnp.float32),
        compiler_params=pltpu.CompilerParams(
            dimension_semantics=("arbitrary",)),
    )(Laplacian, s2, b2r)

    return logits


# speedup vs baseline: 1.0622x; 1.0382x over previous
import jax
import jax.numpy as jnp
from jax.experimental import pallas as pl
from jax.experimental.pallas import tpu as pltpu

_N = 10000
_BM = 400
_NS = _N // _BM


def _fused_kernel(x_ref, w1_ref, b1_ref, w2_ref, b2_ref, l_ref,
                  o_ref, s1_ref, s2_ref):
    i = pl.program_id(0)

    @pl.when(i == 0)
    def _():
        s1_ref[...] = jnp.dot(x_ref[...], w1_ref[...],
                              preferred_element_type=jnp.float32)

    @pl.when((i >= 1) & (i <= _NS))
    def _():
        h = jnp.dot(l_ref[...], s1_ref[...],
                    preferred_element_type=jnp.float32)
        h = jnp.maximum(h + b1_ref[...], 0.0)
        s2_ref[pl.ds((i - 1) * _BM, _BM), :] = jnp.dot(
            h, w2_ref[...], preferred_element_type=jnp.float32)

    @pl.when(i > _NS)
    def _():
        o_ref[...] = jnp.dot(l_ref[...], s2_ref[...],
                             preferred_element_type=jnp.float32) + b2_ref[...]


def _l_stripe(i):
    return (jnp.where(i == 0, 0,
                      jnp.where(i <= _NS, i - 1, i - _NS - 1)), 0)


def _out_stripe(i):
    return (jnp.where(i > _NS, i - _NS - 1, 0), 0)


def kernel(Laplacian, feature, W1, b1, W2, b2):
    n, in_dim = feature.shape
    n_hid = W1.shape[1]
    out_dim = W2.shape[1]
    b1r = b1.reshape(1, n_hid)
    b2r = b2.reshape(1, out_dim)

    return pl.pallas_call(
        _fused_kernel,
        grid=(1 + 2 * _NS,),
        in_specs=[
            pl.BlockSpec((n, in_dim), lambda i: (0, 0)),
            pl.BlockSpec((in_dim, n_hid), lambda i: (0, 0)),
            pl.BlockSpec((1, n_hid), lambda i: (0, 0)),
            pl.BlockSpec((n_hid, out_dim), lambda i: (0, 0)),
            pl.BlockSpec((1, out_dim), lambda i: (0, 0)),
            pl.BlockSpec((_BM, n), _l_stripe),
        ],
        out_specs=pl.BlockSpec((_BM, out_dim), _out_stripe),
        out_shape=jax.ShapeDtypeStruct((n, out_dim), jnp.float32),
        scratch_shapes=[
            pltpu.VMEM((n, n_hid), jnp.float32),
            pltpu.VMEM((n, out_dim), jnp.float32),
        ],
        compiler_params=pltpu.CompilerParams(
            dimension_semantics=("arbitrary",)),
    )(feature, W1, b1r, W2, b2r, Laplacian)
